# Initial kernel scaffold; baseline (speedup 1.0000x reference)
import jax
import jax.numpy as jnp
from jax.experimental import pallas as pl


def kernel(feature_indices, ply, weight, bias):
    raise NotImplementedError("write your pallas kernel here")


# speedup vs baseline: 1.3353x; 1.3353x over previous
import functools

import jax
import jax.numpy as jnp
from jax import lax
from jax.experimental import pallas as pl
from jax.experimental.pallas import tpu as pltpu
from jax.experimental.pallas import tpu_sc as plsc

_NFEAT_TOTAL = 100000
_COUNT = 8
_ODIM = 64
_BUCKET_SIZE = 32
_ACT_SCALE = 255.0 / 256.0
_B = 4096
_F = 50
_FP = 56

_info = plsc.get_sparse_core_info()
_NC = _info.num_cores
_NS = _info.num_subcores
_NW = _NC * _NS
_BPW = _B // _NW
_CH = 8
_NCHUNK = _BPW // _CH


def _body(table, gidx, bucket, bias2, out, idx_v, bucket_v, bias_rows,
          buf, out_v, sem0, sem1):
    wid = lax.axis_index("s") * _NC + lax.axis_index("c")
    base = wid * _BPW

    pltpu.sync_copy(gidx.at[pl.ds(base, _BPW)], idx_v)
    pltpu.sync_copy(bucket.at[pl.ds(base, _BPW)], bucket_v)
    pltpu.async_copy(bias2.at[bucket_v], bias_rows, sem0).wait()

    sems = (sem0, sem1)

    def fire(c, p):
        for j in range(_CH):
            b = c * _CH + j
            pltpu.async_copy(table.at[idx_v.at[b, pl.ds(0, _FP)]],
                             buf.at[p, j], sems[p])

    def drain(c, p):
        for j in range(_CH):
            b = c * _CH + j
            pltpu.make_async_copy(table.at[idx_v.at[b, pl.ds(0, _FP)]],
                                  buf.at[p, j], sems[p]).wait()

    def process(c, p):
        def per_item(j, carry):
            b = c * _CH + j
            for q in range(_ODIM // 16):
                sl = pl.ds(q * 16, 16)
                acc = bias_rows[b, sl]
                for f in range(_F):
                    acc = acc + buf[p, j, f, sl]
                y = jnp.minimum(jnp.maximum(acc, 0.0), 1.0)
                out_v[b, sl] = y * y * jnp.float32(_ACT_SCALE)
            return carry
        lax.fori_loop(0, _CH, per_item, 0)

    fire(0, 0)

    def outer(g, carry):
        c0 = 2 * g
        fire(c0 + 1, 1)
        drain(c0, 0)
        process(c0, 0)

        @pl.when(c0 + 2 < _NCHUNK)
        def _():
            fire(c0 + 2, 0)

        drain(c0 + 1, 1)
        process(c0 + 1, 1)
        return carry

    lax.fori_loop(0, _NCHUNK // 2, outer, 0)
    pltpu.sync_copy(out_v, out.at[pl.ds(base, _BPW)])


@functools.partial(
    pl.kernel,
    out_type=jax.ShapeDtypeStruct((_B, _ODIM), jnp.float32),
    mesh=plsc.VectorSubcoreMesh(core_axis_name="c", subcore_axis_name="s"),
    compiler_params=pltpu.CompilerParams(use_tc_tiling_on_sc=False),
    scratch_types=[
        pltpu.VMEM((_BPW, _FP), jnp.int32),
        pltpu.VMEM((_BPW,), jnp.int32),
        pltpu.VMEM((_BPW, _ODIM), jnp.float32),
        pltpu.VMEM((2, _CH, _FP, _ODIM), jnp.float32),
        pltpu.VMEM((_BPW, _ODIM), jnp.float32),
        pltpu.SemaphoreType.DMA,
        pltpu.SemaphoreType.DMA,
    ],
)
def _gather_sum(table, gidx, bucket, bias2, out, *rest):
    _body(table, gidx, bucket, bias2, out, *rest)


def kernel(feature_indices, ply, weight, bias):
    fi = feature_indices.astype(jnp.int32)
    bkt = ply.astype(jnp.int32) // _BUCKET_SIZE
    gidx = fi * _COUNT + bkt[:, None]
    gidx = jnp.concatenate(
        [gidx, jnp.zeros((_B, _FP - _F), jnp.int32)], axis=1)
    table = weight.reshape(_NFEAT_TOTAL * _COUNT, _ODIM)
    bias2 = bias.reshape(_COUNT, _ODIM)
    return _gather_sum(table, gidx, bkt, bias2)
